# Initial kernel scaffold; baseline (speedup 1.0000x reference)
import jax
import jax.numpy as jnp
from jax.experimental import pallas as pl


def kernel(atomic_numbers, ref_energy_weight):
    raise NotImplementedError("write your pallas kernel here")


# speedup vs baseline: 405.5644x; 405.5644x over previous
import dataclasses
import functools

import jax
import jax.numpy as jnp
from jax import lax
from jax.experimental import pallas as pl
from jax.experimental.pallas import tpu as pltpu
from jax.experimental.pallas import tpu_sc as plsc

B = 16384
LROW = 200
NCORES = 2
NSUB = 16
NW = NCORES * NSUB
ROWS_PER_W = B // NW
ELEMS_PER_W = ROWS_PER_W * LROW
PAIRS = ROWS_PER_W // 2
LANES = 16
VECS_PER_PAIR = 2 * LROW // LANES


def _sc_rowsum(flat_idx, table128):
    mesh = plsc.VectorSubcoreMesh(core_axis_name="c", subcore_axis_name="s")
    cp = pltpu.CompilerParams()
    if "needs_layout_passes" in pltpu.CompilerParams.__dataclass_fields__:
        cp = dataclasses.replace(cp, needs_layout_passes=False)

    @functools.partial(
        pl.kernel,
        compiler_params=cp,
        out_type=jax.ShapeDtypeStruct((B,), jnp.float32),
        mesh=mesh,
        scratch_types=[
            pltpu.VMEM((128,), jnp.float32),
            pltpu.VMEM((ELEMS_PER_W,), jnp.int32),
            pltpu.VMEM((ROWS_PER_W,), jnp.float32),
        ],
    )
    def k(flat_hbm, table_hbm, out_hbm, table_v, block_v, out_v):
        wid = lax.axis_index("s") * NCORES + lax.axis_index("c")
        pltpu.sync_copy(table_hbm, table_v)
        pltpu.sync_copy(
            flat_hbm.at[pl.ds(wid * ELEMS_PER_W, ELEMS_PER_W)], block_v
        )

        lane = lax.iota(jnp.int32, LANES)
        mask_lo = lane < (LROW % LANES)

        @pl.loop(0, PAIRS // 8)
        def _(g):
            acc_out = jnp.zeros((LANES,), jnp.float32)
            for jp in range(8):
                base = g * (16 * LROW) + jp * (2 * LROW)
                acc_a = jnp.zeros((LANES,), jnp.float32)
                acc_b = jnp.zeros((LANES,), jnp.float32)
                for kv in range(12):
                    idx = block_v[pl.ds(base + kv * LANES, LANES)]
                    acc_a = acc_a + plsc.load_gather(table_v, [idx])
                idx = block_v[pl.ds(base + 12 * LANES, LANES)]
                v = plsc.load_gather(table_v, [idx])
                acc_a = acc_a + jnp.where(mask_lo, v, 0.0)
                acc_b = acc_b + jnp.where(mask_lo, 0.0, v)
                for kv in range(13, VECS_PER_PAIR):
                    idx = block_v[pl.ds(base + kv * LANES, LANES)]
                    acc_b = acc_b + plsc.load_gather(table_v, [idx])
                acc_out = jnp.where(lane == 2 * jp, jnp.sum(acc_a), acc_out)
                acc_out = jnp.where(lane == 2 * jp + 1, jnp.sum(acc_b), acc_out)
            out_v[pl.ds(g * LANES, LANES)] = acc_out

        pltpu.sync_copy(out_v, out_hbm.at[pl.ds(wid * ROWS_PER_W, ROWS_PER_W)])

    return k(flat_idx, table128)


@jax.jit
def kernel(atomic_numbers, ref_energy_weight):
    flat_idx = atomic_numbers.reshape(-1)
    table128 = jnp.zeros((128,), jnp.float32).at[:100].set(
        ref_energy_weight[:, 0]
    )
    return _sc_rowsum(flat_idx, table128)
